# baseline (speedup 1.0000x reference)
import functools

import jax
import jax.numpy as jnp
from jax import lax
from jax.experimental import pallas as pl
from jax.experimental.pallas import tpu as pltpu
from jax.experimental.pallas import tpu_sc as plsc

N = 10000
E = 320000
D = 128

NC = 2
NS = 16
NW = NC * NS
K = 80
C = E // (NW * K)
K2 = 48
C2 = 212
EPAD = NW * C2 * K2 - E
PAD_NODE = N
HW = 16
NP = 10240
RT = NP // NS

_mesh = plsc.VectorSubcoreMesh(core_axis_name="c", subcore_axis_name="s")


@functools.partial(
    pl.kernel,
    out_type=jax.ShapeDtypeStruct((NC, NP, HW), jnp.float32),
    mesh=_mesh,
    compiler_params=pltpu.CompilerParams(use_tc_tiling_on_sc=False),
    scratch_types=[
        pltpu.VMEM((C, K), jnp.int32),
        pltpu.VMEM((K, HW), jnp.float32),
        pltpu.VMEM_SHARED((NP, HW), jnp.float32),
    ],
)
def _degree_hist(dst_hbm, ones_hbm, zeros_hbm, out_hbm, idx_v, ones_v, hist_sh):
    cid = lax.axis_index("c")
    sid = lax.axis_index("s")
    wid = sid * NC + cid
    pltpu.sync_copy(zeros_hbm.at[pl.ds(sid * RT, RT)],
                    hist_sh.at[pl.ds(sid * RT, RT)])
    pltpu.sync_copy(ones_hbm, ones_v)
    pltpu.sync_copy(dst_hbm.at[wid], idx_v)
    plsc.subcore_barrier()

    def step(c, carry):
        pltpu.sync_copy(ones_v, hist_sh.at[idx_v.at[c]], add=True)
        return carry

    lax.fori_loop(0, C, step, 0)
    plsc.subcore_barrier()
    pltpu.sync_copy(hist_sh.at[pl.ds(sid * RT, RT)],
                    out_hbm.at[cid, pl.ds(sid * RT, RT), :])


@functools.partial(
    pl.kernel,
    out_type=jax.ShapeDtypeStruct((NC, NP, D), jnp.float32),
    mesh=_mesh,
    compiler_params=pltpu.CompilerParams(use_tc_tiling_on_sc=False),
    scratch_types=[
        pltpu.VMEM((C2, K2), jnp.int32),
        pltpu.VMEM((C2, K2), jnp.int32),
        pltpu.VMEM((K2, D), jnp.float32),
        pltpu.VMEM((K2, D), jnp.float32),
        pltpu.VMEM((K2, D), jnp.float32),
        pltpu.VMEM((K2, D), jnp.float32),
        pltpu.VMEM_SHARED((NP, D), jnp.float32),
        pltpu.SemaphoreType.DMA,
        pltpu.SemaphoreType.DMA,
        pltpu.SemaphoreType.DMA,
        pltpu.SemaphoreType.DMA,
    ],
)
def _aggregate(src_hbm, dst_hbm, hs_hbm, zeros_hbm, out_hbm, src_v, dst_v,
               rows0, rows1, rows2, rows3, acc_sh, sem0, sem1, sem2, sem3):
    cid = lax.axis_index("c")
    sid = lax.axis_index("s")
    wid = sid * NC + cid
    pltpu.sync_copy(zeros_hbm.at[pl.ds(sid * RT, RT)],
                    acc_sh.at[pl.ds(sid * RT, RT)])
    pltpu.sync_copy(src_hbm.at[wid], src_v)
    pltpu.sync_copy(dst_hbm.at[wid], dst_v)
    plsc.subcore_barrier()

    def gstart(c, rows, sem):
        pltpu.async_copy(hs_hbm.at[src_v.at[c]], rows, sem)

    def gwait(c, rows, sem):
        pltpu.make_async_copy(hs_hbm.at[src_v.at[c]], rows, sem).wait()

    def scat(c, rows):
        pltpu.sync_copy(rows, acc_sh.at[dst_v.at[c]], add=True)

    bufs = ((rows0, sem0), (rows1, sem1), (rows2, sem2), (rows3, sem3))
    RD = len(bufs)
    for j, (rows, sem) in enumerate(bufs):
        gstart(j, rows, sem)

    def step(i, carry):
        c = RD * i
        for j, (rows, sem) in enumerate(bufs):
            gwait(c + j, rows, sem)

            @pl.when(c + j + RD < C2)
            def _():
                gstart(c + j + RD, rows, sem)

            scat(c + j, rows)
        return carry

    lax.fori_loop(0, C2 // RD, step, 0)
    plsc.subcore_barrier()
    pltpu.sync_copy(acc_sh.at[pl.ds(sid * RT, RT)],
                    out_hbm.at[cid, pl.ds(sid * RT, RT), :])


BN = 1000


def _scaled_mm_body(x_ref, w_ref, b_ref, h_ref, out_ref):
    h = h_ref[...]
    deg = h[0, :, 0] + h[1, :, 0] + 1.0
    dis = lax.rsqrt(deg)
    y = jnp.dot(x_ref[...], w_ref[...], preferred_element_type=jnp.float32)
    out_ref[...] = dis[:, None] * (y + b_ref[...])


def _epilogue_body(a0_ref, a1_ref, hs_ref, h_ref, out_ref):
    h = h_ref[...]
    deg = h[0, :, 0] + h[1, :, 0] + 1.0
    dis = lax.rsqrt(deg)
    s = a0_ref[...] + a1_ref[...] + hs_ref[...]
    out_ref[...] = jnp.maximum(dis[:, None] * s, 0.0)


def kernel(node_feats, adj, is_training, w, b):
    del is_training
    dst3 = adj[1].reshape(NW, C, K)
    pad = PAD_NODE + (jnp.arange(EPAD, dtype=jnp.int32) % (NP - N))
    srcp = jnp.concatenate([adj[0], pad]).reshape(NW, C2, K2)
    dstp = jnp.concatenate([adj[1], pad]).reshape(NW, C2, K2)
    ones = jnp.ones((K, HW), jnp.float32)
    zeros = jnp.zeros((NP, D), jnp.float32)
    zeros_h = jnp.zeros((NP, HW), jnp.float32)

    hist = _degree_hist(dst3, ones, zeros_h)

    hs = pl.pallas_call(
        _scaled_mm_body,
        grid=(N // BN,),
        in_specs=[
            pl.BlockSpec((BN, D), lambda i: (i, 0)),
            pl.BlockSpec((D, D), lambda i: (0, 0)),
            pl.BlockSpec((1, D), lambda i: (0, 0)),
            pl.BlockSpec((NC, BN, HW), lambda i: (0, i, 0)),
        ],
        out_specs=pl.BlockSpec((BN, D), lambda i: (i, 0)),
        out_shape=jax.ShapeDtypeStruct((NP, D), jnp.float32),
    )(node_feats, w, b.reshape(1, D), hist)

    acc = _aggregate(srcp, dstp, hs, zeros)

    out = pl.pallas_call(
        _epilogue_body,
        grid=(N // BN,),
        in_specs=[
            pl.BlockSpec((BN, D), lambda i: (i, 0)),
            pl.BlockSpec((BN, D), lambda i: (i, 0)),
            pl.BlockSpec((BN, D), lambda i: (i, 0)),
            pl.BlockSpec((NC, BN, HW), lambda i: (0, i, 0)),
        ],
        out_specs=pl.BlockSpec((BN, D), lambda i: (i, 0)),
        out_shape=jax.ShapeDtypeStruct((N, D), jnp.float32),
    )(acc[0], acc[1], hs, hist)
    return out


# speedup vs baseline: 1.0432x; 1.0432x over previous
import functools

import jax
import jax.numpy as jnp
from jax import lax
from jax.experimental import pallas as pl
from jax.experimental.pallas import tpu as pltpu
from jax.experimental.pallas import tpu_sc as plsc

N = 10000
E = 320000
D = 128

NC = 2
NS = 16
NW = NC * NS
K = 80
C = E // (NW * K)
K2 = 48
C2 = 212
EPAD = NW * C2 * K2 - E
PAD_NODE = N
HW = 16
NP = 10240
RT = NP // NS

_mesh = plsc.VectorSubcoreMesh(core_axis_name="c", subcore_axis_name="s")


@functools.partial(
    pl.kernel,
    out_type=jax.ShapeDtypeStruct((NC, NP, HW), jnp.float32),
    mesh=_mesh,
    compiler_params=pltpu.CompilerParams(use_tc_tiling_on_sc=False),
    scratch_types=[
        pltpu.VMEM((C, K), jnp.int32),
        pltpu.VMEM((K, HW), jnp.float32),
        pltpu.VMEM_SHARED((NP, HW), jnp.float32),
        pltpu.SemaphoreType.DMA,
    ],
)
def _degree_hist(dst_hbm, ones_hbm, zeros_hbm, out_hbm, idx_v, ones_v,
                 hist_sh, sem):
    cid = lax.axis_index("c")
    sid = lax.axis_index("s")
    wid = sid * NC + cid
    pltpu.sync_copy(zeros_hbm.at[pl.ds(sid * RT, RT)],
                    hist_sh.at[pl.ds(sid * RT, RT)])
    pltpu.sync_copy(ones_hbm, ones_v)
    pltpu.sync_copy(dst_hbm.at[wid], idx_v)
    plsc.subcore_barrier()

    W = 8
    for c in range(W):
        pltpu.async_copy(ones_v, hist_sh.at[idx_v.at[c]], sem, add=True)

    def step(c, carry):
        pltpu.make_async_copy(ones_v, hist_sh.at[idx_v.at[0]], sem).wait()
        pltpu.async_copy(ones_v, hist_sh.at[idx_v.at[c + W]], sem, add=True)
        return carry

    lax.fori_loop(0, C - W, step, 0)
    for c in range(W):
        pltpu.make_async_copy(ones_v, hist_sh.at[idx_v.at[0]], sem).wait()
    plsc.subcore_barrier()
    pltpu.sync_copy(hist_sh.at[pl.ds(sid * RT, RT)],
                    out_hbm.at[cid, pl.ds(sid * RT, RT), :])


@functools.partial(
    pl.kernel,
    out_type=jax.ShapeDtypeStruct((NC, NP, D), jnp.float32),
    mesh=_mesh,
    compiler_params=pltpu.CompilerParams(use_tc_tiling_on_sc=False),
    scratch_types=[
        pltpu.VMEM((C2, K2), jnp.int32),
        pltpu.VMEM((C2, K2), jnp.int32),
        pltpu.VMEM((K2, D), jnp.float32),
        pltpu.VMEM((K2, D), jnp.float32),
        pltpu.VMEM((K2, D), jnp.float32),
        pltpu.VMEM((K2, D), jnp.float32),
        pltpu.VMEM_SHARED((NP, D), jnp.float32),
        pltpu.SemaphoreType.DMA,
        pltpu.SemaphoreType.DMA,
        pltpu.SemaphoreType.DMA,
        pltpu.SemaphoreType.DMA,
    ],
)
def _aggregate(src_hbm, dst_hbm, hs_hbm, zeros_hbm, out_hbm, src_v, dst_v,
               rows0, rows1, rows2, rows3, acc_sh, sem0, sem1, sem2, sem3):
    cid = lax.axis_index("c")
    sid = lax.axis_index("s")
    wid = sid * NC + cid
    @pl.when(cid == 0)
    def _():
        pltpu.sync_copy(hs_hbm.at[pl.ds(sid * RT, RT)],
                        acc_sh.at[pl.ds(sid * RT, RT)])

    @pl.when(cid == 1)
    def _():
        pltpu.sync_copy(zeros_hbm.at[pl.ds(sid * RT, RT)],
                        acc_sh.at[pl.ds(sid * RT, RT)])

    pltpu.sync_copy(src_hbm.at[wid], src_v)
    pltpu.sync_copy(dst_hbm.at[wid], dst_v)
    plsc.subcore_barrier()

    def gstart(c, rows, sem):
        pltpu.async_copy(hs_hbm.at[src_v.at[c]], rows, sem)

    def gwait(c, rows, sem):
        pltpu.make_async_copy(hs_hbm.at[src_v.at[c]], rows, sem).wait()

    def scat(c, rows):
        pltpu.sync_copy(rows, acc_sh.at[dst_v.at[c]], add=True)

    bufs = ((rows0, sem0), (rows1, sem1), (rows2, sem2), (rows3, sem3))
    RD = len(bufs)
    for j, (rows, sem) in enumerate(bufs):
        gstart(j, rows, sem)

    def step(i, carry):
        c = RD * i
        for j, (rows, sem) in enumerate(bufs):
            gwait(c + j, rows, sem)

            @pl.when(c + j + RD < C2)
            def _():
                gstart(c + j + RD, rows, sem)

            scat(c + j, rows)
        return carry

    lax.fori_loop(0, C2 // RD, step, 0)
    plsc.subcore_barrier()
    pltpu.sync_copy(acc_sh.at[pl.ds(sid * RT, RT)],
                    out_hbm.at[cid, pl.ds(sid * RT, RT), :])


BN = 1000


def _scaled_mm_body(x_ref, w_ref, b_ref, h_ref, out_ref):
    h = h_ref[...]
    deg = h[0, :, 0] + h[1, :, 0] + 1.0
    dis = lax.rsqrt(deg)
    y = jnp.dot(x_ref[...], w_ref[...], preferred_element_type=jnp.float32)
    out_ref[...] = dis[:, None] * (y + b_ref[...])


def _epilogue_body(a0_ref, a1_ref, h_ref, out_ref):
    h = h_ref[...]
    deg = h[0, :, 0] + h[1, :, 0] + 1.0
    dis = lax.rsqrt(deg)
    s = a0_ref[...] + a1_ref[...]
    out_ref[...] = jnp.maximum(dis[:, None] * s, 0.0)


def kernel(node_feats, adj, is_training, w, b):
    del is_training
    dst3 = adj[1].reshape(NW, C, K)
    pad = PAD_NODE + (jnp.arange(EPAD, dtype=jnp.int32) % (NP - N))
    srcp = jnp.concatenate([adj[0], pad]).reshape(NW, C2, K2)
    dstp = jnp.concatenate([adj[1], pad]).reshape(NW, C2, K2)
    ones = jnp.ones((K, HW), jnp.float32)
    zeros = jnp.zeros((NP, D), jnp.float32)
    zeros_h = jnp.zeros((NP, HW), jnp.float32)

    hist = _degree_hist(dst3, ones, zeros_h)

    hs = pl.pallas_call(
        _scaled_mm_body,
        grid=(N // BN,),
        in_specs=[
            pl.BlockSpec((BN, D), lambda i: (i, 0)),
            pl.BlockSpec((D, D), lambda i: (0, 0)),
            pl.BlockSpec((1, D), lambda i: (0, 0)),
            pl.BlockSpec((NC, BN, HW), lambda i: (0, i, 0)),
        ],
        out_specs=pl.BlockSpec((BN, D), lambda i: (i, 0)),
        out_shape=jax.ShapeDtypeStruct((NP, D), jnp.float32),
    )(node_feats, w, b.reshape(1, D), hist)

    acc = _aggregate(srcp, dstp, hs, zeros)

    out = pl.pallas_call(
        _epilogue_body,
        grid=(N // BN,),
        in_specs=[
            pl.BlockSpec((BN, D), lambda i: (i, 0)),
            pl.BlockSpec((BN, D), lambda i: (i, 0)),
            pl.BlockSpec((NC, BN, HW), lambda i: (0, i, 0)),
        ],
        out_specs=pl.BlockSpec((BN, D), lambda i: (i, 0)),
        out_shape=jax.ShapeDtypeStruct((N, D), jnp.float32),
    )(acc[0], acc[1], hist)
    return out


# speedup vs baseline: 1.0657x; 1.0215x over previous
import functools

import jax
import jax.numpy as jnp
from jax import lax
from jax.experimental import pallas as pl
from jax.experimental.pallas import tpu as pltpu
from jax.experimental.pallas import tpu_sc as plsc

N = 10000
E = 320000
D = 128

NC = 2
NS = 16
NW = NC * NS
K = 128
C = 79
EPADH = NW * C * K - E
K2 = 48
C2 = 212
EPAD = NW * C2 * K2 - E
PAD_NODE = N
HW = 16
NP = 10240
RT = NP // NS

_mesh = plsc.VectorSubcoreMesh(core_axis_name="c", subcore_axis_name="s")


@functools.partial(
    pl.kernel,
    out_type=jax.ShapeDtypeStruct((NC, NP, HW), jnp.float32),
    mesh=_mesh,
    compiler_params=pltpu.CompilerParams(use_tc_tiling_on_sc=False),
    scratch_types=[
        pltpu.VMEM((C, K), jnp.int32),
        pltpu.VMEM((K, HW), jnp.float32),
        pltpu.VMEM((K, HW), jnp.float32),
        pltpu.VMEM_SHARED((NP, HW), jnp.float32),
        pltpu.SemaphoreType.DMA,
    ],
)
def _degree_hist(dst_hbm, ones_hbm, out_hbm, idx_v, ones_v, zero_v,
                 hist_sh, sem):
    cid = lax.axis_index("c")
    sid = lax.axis_index("s")
    wid = sid * NC + cid
    zero16 = jnp.zeros((16,), jnp.float32)

    def zfill(t, carry):
        zero_v[t, :] = zero16
        return carry

    lax.fori_loop(0, K, zfill, 0)
    for j in range(RT // K):
        pltpu.sync_copy(zero_v, hist_sh.at[pl.ds(sid * RT + j * K, K)])
    pltpu.sync_copy(ones_hbm, ones_v)
    pltpu.sync_copy(dst_hbm.at[wid], idx_v)
    plsc.subcore_barrier()

    W = 8
    for c in range(W):
        pltpu.async_copy(ones_v, hist_sh.at[idx_v.at[c]], sem, add=True)

    def step(c, carry):
        pltpu.make_async_copy(ones_v, hist_sh.at[idx_v.at[0]], sem).wait()
        pltpu.async_copy(ones_v, hist_sh.at[idx_v.at[c + W]], sem, add=True)
        return carry

    lax.fori_loop(0, C - W, step, 0)
    for c in range(W):
        pltpu.make_async_copy(ones_v, hist_sh.at[idx_v.at[0]], sem).wait()
    plsc.subcore_barrier()
    pltpu.sync_copy(hist_sh.at[pl.ds(sid * RT, RT)],
                    out_hbm.at[cid, pl.ds(sid * RT, RT), :])


@functools.partial(
    pl.kernel,
    out_type=jax.ShapeDtypeStruct((NC, NP, D), jnp.float32),
    mesh=_mesh,
    compiler_params=pltpu.CompilerParams(use_tc_tiling_on_sc=False),
    scratch_types=[
        pltpu.VMEM((C2, K2), jnp.int32),
        pltpu.VMEM((C2, K2), jnp.int32),
        pltpu.VMEM((K2, D), jnp.float32),
        pltpu.VMEM((K2, D), jnp.float32),
        pltpu.VMEM((K2, D), jnp.float32),
        pltpu.VMEM((K2, D), jnp.float32),
        pltpu.VMEM_SHARED((NP, D), jnp.float32),
        pltpu.SemaphoreType.DMA,
        pltpu.SemaphoreType.DMA,
        pltpu.SemaphoreType.DMA,
        pltpu.SemaphoreType.DMA,
    ],
)
def _aggregate(src_hbm, dst_hbm, hs_hbm, out_hbm, src_v, dst_v,
               rows0, rows1, rows2, rows3, acc_sh, sem0, sem1, sem2, sem3):
    cid = lax.axis_index("c")
    sid = lax.axis_index("s")
    wid = sid * NC + cid
    @pl.when(cid == 0)
    def _():
        pltpu.sync_copy(hs_hbm.at[pl.ds(sid * RT, RT)],
                        acc_sh.at[pl.ds(sid * RT, RT)])

    @pl.when(cid == 1)
    def _():
        zero16 = jnp.zeros((16,), jnp.float32)

        def zfill(t, carry):
            rows0[t // 8, pl.ds((t % 8) * 16, 16)] = zero16
            return carry

        lax.fori_loop(0, K2 * 8, zfill, 0)
        for j in range(RT // K2):
            pltpu.sync_copy(rows0, acc_sh.at[pl.ds(sid * RT + j * K2, K2)])
        rem = RT - (RT // K2) * K2
        if rem:
            pltpu.sync_copy(rows0.at[pl.ds(0, rem)],
                            acc_sh.at[pl.ds(sid * RT + RT - rem, rem)])

    pltpu.sync_copy(src_hbm.at[wid], src_v)
    pltpu.sync_copy(dst_hbm.at[wid], dst_v)
    plsc.subcore_barrier()

    def gstart(c, rows, sem):
        pltpu.async_copy(hs_hbm.at[src_v.at[c]], rows, sem)

    def gwait(c, rows, sem):
        pltpu.make_async_copy(hs_hbm.at[src_v.at[c]], rows, sem).wait()

    def scat(c, rows):
        pltpu.sync_copy(rows, acc_sh.at[dst_v.at[c]], add=True)

    bufs = ((rows0, sem0), (rows1, sem1), (rows2, sem2), (rows3, sem3))
    RD = len(bufs)
    for j, (rows, sem) in enumerate(bufs):
        gstart(j, rows, sem)

    def step(i, carry):
        c = RD * i
        for j, (rows, sem) in enumerate(bufs):
            gwait(c + j, rows, sem)

            @pl.when(c + j + RD < C2)
            def _():
                gstart(c + j + RD, rows, sem)

            scat(c + j, rows)
        return carry

    lax.fori_loop(0, C2 // RD, step, 0)
    plsc.subcore_barrier()
    pltpu.sync_copy(acc_sh.at[pl.ds(sid * RT, RT)],
                    out_hbm.at[cid, pl.ds(sid * RT, RT), :])


BN = 1000


def _scaled_mm_body(x_ref, w_ref, b_ref, h_ref, out_ref):
    h = h_ref[...]
    deg = h[0, :, 0] + h[1, :, 0] + 1.0
    dis = lax.rsqrt(deg)
    y = jnp.dot(x_ref[...], w_ref[...], preferred_element_type=jnp.float32)
    out_ref[...] = dis[:, None] * (y + b_ref[...])


def _epilogue_body(a0_ref, a1_ref, h_ref, out_ref):
    h = h_ref[...]
    deg = h[0, :, 0] + h[1, :, 0] + 1.0
    dis = lax.rsqrt(deg)
    s = a0_ref[...] + a1_ref[...]
    out_ref[...] = jnp.maximum(dis[:, None] * s, 0.0)


def kernel(node_feats, adj, is_training, w, b):
    del is_training
    padh = PAD_NODE + (jnp.arange(EPADH, dtype=jnp.int32) % (NP - N))
    dst3 = jnp.concatenate([adj[1], padh]).reshape(NW, C, K)
    pad = PAD_NODE + (jnp.arange(EPAD, dtype=jnp.int32) % (NP - N))
    srcp = jnp.concatenate([adj[0], pad]).reshape(NW, C2, K2)
    dstp = jnp.concatenate([adj[1], pad]).reshape(NW, C2, K2)
    ones = jnp.ones((K, HW), jnp.float32)

    hist = _degree_hist(dst3, ones)

    hs = pl.pallas_call(
        _scaled_mm_body,
        grid=(N // BN,),
        in_specs=[
            pl.BlockSpec((BN, D), lambda i: (i, 0)),
            pl.BlockSpec((D, D), lambda i: (0, 0)),
            pl.BlockSpec((1, D), lambda i: (0, 0)),
            pl.BlockSpec((NC, BN, HW), lambda i: (0, i, 0)),
        ],
        out_specs=pl.BlockSpec((BN, D), lambda i: (i, 0)),
        out_shape=jax.ShapeDtypeStruct((NP, D), jnp.float32),
    )(node_feats, w, b.reshape(1, D), hist)

    acc = _aggregate(srcp, dstp, hs)

    out = pl.pallas_call(
        _epilogue_body,
        grid=(N // BN,),
        in_specs=[
            pl.BlockSpec((BN, D), lambda i: (i, 0)),
            pl.BlockSpec((BN, D), lambda i: (i, 0)),
            pl.BlockSpec((NC, BN, HW), lambda i: (0, i, 0)),
        ],
        out_specs=pl.BlockSpec((BN, D), lambda i: (i, 0)),
        out_shape=jax.ShapeDtypeStruct((N, D), jnp.float32),
    )(acc[0], acc[1], hist)
    return out


# speedup vs baseline: 1.0939x; 1.0265x over previous
import functools

import jax
import jax.numpy as jnp
from jax import lax
from jax.experimental import pallas as pl
from jax.experimental.pallas import tpu as pltpu
from jax.experimental.pallas import tpu_sc as plsc

N = 10000
E = 320000
D = 128

NC = 2
NS = 16
NW = NC * NS
K = 128
C = 79
EPADH = NW * C * K - E
K2 = 48
C2 = 212
EPAD = NW * C2 * K2 - E
PAD_NODE = N
HW = 16
NP = 10240
RT = NP // NS

_mesh = plsc.VectorSubcoreMesh(core_axis_name="c", subcore_axis_name="s")


@functools.partial(
    pl.kernel,
    out_type=jax.ShapeDtypeStruct((NC, NP, HW), jnp.float32),
    mesh=_mesh,
    compiler_params=pltpu.CompilerParams(use_tc_tiling_on_sc=False),
    scratch_types=[
        pltpu.VMEM((C, K), jnp.int32),
        pltpu.VMEM((K, HW), jnp.float32),
        pltpu.VMEM((K, HW), jnp.float32),
        pltpu.VMEM_SHARED((NP, HW), jnp.float32),
        pltpu.SemaphoreType.DMA,
    ],
)
def _degree_hist(dst_hbm, ones_hbm, out_hbm, idx_v, ones_v, zero_v,
                 hist_sh, sem):
    cid = lax.axis_index("c")
    sid = lax.axis_index("s")
    wid = sid * NC + cid
    zero16 = jnp.zeros((16,), jnp.float32)

    def zfill(t, carry):
        zero_v[t, :] = zero16
        return carry

    lax.fori_loop(0, K, zfill, 0)
    for j in range(RT // K):
        pltpu.sync_copy(zero_v, hist_sh.at[pl.ds(sid * RT + j * K, K)])
    pltpu.sync_copy(ones_hbm, ones_v)
    pltpu.sync_copy(dst_hbm.at[wid], idx_v)
    plsc.subcore_barrier()

    W = 8
    for c in range(W):
        pltpu.async_copy(ones_v, hist_sh.at[idx_v.at[c]], sem, add=True)

    def step(c, carry):
        pltpu.make_async_copy(ones_v, hist_sh.at[idx_v.at[0]], sem).wait()
        pltpu.async_copy(ones_v, hist_sh.at[idx_v.at[c + W]], sem, add=True)
        return carry

    lax.fori_loop(0, C - W, step, 0)
    for c in range(W):
        pltpu.make_async_copy(ones_v, hist_sh.at[idx_v.at[0]], sem).wait()
    plsc.subcore_barrier()
    pltpu.sync_copy(hist_sh.at[pl.ds(sid * RT, RT)],
                    out_hbm.at[cid, pl.ds(sid * RT, RT), :])


@functools.partial(
    pl.kernel,
    out_type=jax.ShapeDtypeStruct((NC, NP, D), jnp.float32),
    mesh=_mesh,
    compiler_params=pltpu.CompilerParams(use_tc_tiling_on_sc=False),
    scratch_types=[
        pltpu.VMEM((C2, K2), jnp.int32),
        pltpu.VMEM((C2, K2), jnp.int32),
        pltpu.VMEM((K2, D), jnp.float32),
        pltpu.VMEM((K2, D), jnp.float32),
        pltpu.VMEM((K2, D), jnp.float32),
        pltpu.VMEM((K2, D), jnp.float32),
        pltpu.VMEM_SHARED((NP, D), jnp.float32),
        pltpu.SemaphoreType.DMA,
        pltpu.SemaphoreType.DMA,
        pltpu.SemaphoreType.DMA,
        pltpu.SemaphoreType.DMA,
    ],
)
def _aggregate(src_hbm, dst_hbm, hs_hbm, out_hbm, src_v, dst_v,
               rows0, rows1, rows2, rows3, acc_sh, sem0, sem1, sem2, sem3):
    cid = lax.axis_index("c")
    sid = lax.axis_index("s")
    wid = sid * NC + cid
    @pl.when(cid == 0)
    def _():
        pltpu.sync_copy(hs_hbm.at[pl.ds(sid * RT, RT)],
                        acc_sh.at[pl.ds(sid * RT, RT)])

    @pl.when(cid == 1)
    def _():
        zero16 = jnp.zeros((16,), jnp.float32)

        def zfill(t, carry):
            rows0[t // 8, pl.ds((t % 8) * 16, 16)] = zero16
            return carry

        lax.fori_loop(0, K2 * 8, zfill, 0)
        for j in range(RT // K2):
            pltpu.sync_copy(rows0, acc_sh.at[pl.ds(sid * RT + j * K2, K2)])
        rem = RT - (RT // K2) * K2
        if rem:
            pltpu.sync_copy(rows0.at[pl.ds(0, rem)],
                            acc_sh.at[pl.ds(sid * RT + RT - rem, rem)])

    pltpu.sync_copy(src_hbm.at[wid], src_v)
    pltpu.sync_copy(dst_hbm.at[wid], dst_v)
    plsc.subcore_barrier()

    def gstart(c, rows, sem):
        pltpu.async_copy(hs_hbm.at[src_v.at[c]], rows, sem)

    def gwait(c, rows, sem):
        pltpu.make_async_copy(hs_hbm.at[src_v.at[c]], rows, sem).wait()

    def scat(c, rows):
        pltpu.sync_copy(rows, acc_sh.at[dst_v.at[c]], add=True)

    bufs = ((rows0, sem0), (rows1, sem1), (rows2, sem2), (rows3, sem3))
    RD = len(bufs)
    for j, (rows, sem) in enumerate(bufs):
        gstart(j, rows, sem)

    def step(i, carry):
        c = RD * i
        for j, (rows, sem) in enumerate(bufs):
            gwait(c + j, rows, sem)

            @pl.when(c + j + RD < C2)
            def _():
                gstart(c + j + RD, rows, sem)

            scat(c + j, rows)
        return carry

    lax.fori_loop(0, C2 // RD, step, 0)
    plsc.subcore_barrier()
    pltpu.sync_copy(acc_sh.at[pl.ds(sid * RT, RT)],
                    out_hbm.at[cid, pl.ds(sid * RT, RT), :])


BN = 2000


def _scaled_mm_body(x_ref, w_ref, b_ref, h_ref, out_ref):
    h = h_ref[...]
    deg = h[0, :, 0] + h[1, :, 0] + 1.0
    dis = lax.rsqrt(deg)
    y = jnp.dot(x_ref[...], w_ref[...], preferred_element_type=jnp.float32)
    out_ref[...] = dis[:, None] * (y + b_ref[...])


def _epilogue_body(a0_ref, a1_ref, h_ref, out_ref):
    h = h_ref[...]
    deg = h[0, :, 0] + h[1, :, 0] + 1.0
    dis = lax.rsqrt(deg)
    s = a0_ref[...] + a1_ref[...]
    out_ref[...] = jnp.maximum(dis[:, None] * s, 0.0)


def kernel(node_feats, adj, is_training, w, b):
    del is_training
    padh = PAD_NODE + (jnp.arange(EPADH, dtype=jnp.int32) % (NP - N))
    dst3 = jnp.concatenate([adj[1], padh]).reshape(NW, C, K)
    pad = PAD_NODE + (jnp.arange(EPAD, dtype=jnp.int32) % (NP - N))
    srcp = jnp.concatenate([adj[0], pad]).reshape(NW, C2, K2)
    dstp = jnp.concatenate([adj[1], pad]).reshape(NW, C2, K2)
    ones = jnp.ones((K, HW), jnp.float32)

    hist = _degree_hist(dst3, ones)

    hs = pl.pallas_call(
        _scaled_mm_body,
        grid=(N // BN,),
        in_specs=[
            pl.BlockSpec((BN, D), lambda i: (i, 0)),
            pl.BlockSpec((D, D), lambda i: (0, 0)),
            pl.BlockSpec((1, D), lambda i: (0, 0)),
            pl.BlockSpec((NC, BN, HW), lambda i: (0, i, 0)),
        ],
        out_specs=pl.BlockSpec((BN, D), lambda i: (i, 0)),
        out_shape=jax.ShapeDtypeStruct((NP, D), jnp.float32),
    )(node_feats, w, b.reshape(1, D), hist)

    acc = _aggregate(srcp, dstp, hs)

    out = pl.pallas_call(
        _epilogue_body,
        grid=(N // BN,),
        in_specs=[
            pl.BlockSpec((BN, D), lambda i: (i, 0)),
            pl.BlockSpec((BN, D), lambda i: (i, 0)),
            pl.BlockSpec((NC, BN, HW), lambda i: (0, i, 0)),
        ],
        out_specs=pl.BlockSpec((BN, D), lambda i: (i, 0)),
        out_shape=jax.ShapeDtypeStruct((N, D), jnp.float32),
    )(acc[0], acc[1], hist)
    return out


# speedup vs baseline: 1.1012x; 1.0067x over previous
import functools

import jax
import jax.numpy as jnp
from jax import lax
from jax.experimental import pallas as pl
from jax.experimental.pallas import tpu as pltpu
from jax.experimental.pallas import tpu_sc as plsc

N = 10000
E = 320000
D = 128

NC = 2
NS = 16
NW = NC * NS
K = 128
C = 79
EPADH = NW * C * K - E
K2 = 48
C2 = 212
EPAD = NW * C2 * K2 - E
PAD_NODE = N
HW = 16
NP = 10240
RT = NP // NS

_mesh = plsc.VectorSubcoreMesh(core_axis_name="c", subcore_axis_name="s")


@functools.partial(
    pl.kernel,
    out_type=jax.ShapeDtypeStruct((NC, NP, HW), jnp.float32),
    mesh=_mesh,
    compiler_params=pltpu.CompilerParams(use_tc_tiling_on_sc=False),
    scratch_types=[
        pltpu.VMEM((C, K), jnp.int32),
        pltpu.VMEM((K, HW), jnp.float32),
        pltpu.VMEM((K, HW), jnp.float32),
        pltpu.VMEM_SHARED((NP, HW), jnp.float32),
        pltpu.SemaphoreType.DMA,
    ],
)
def _degree_hist(dst_hbm, ones_hbm, out_hbm, idx_v, ones_v, zero_v,
                 hist_sh, sem):
    cid = lax.axis_index("c")
    sid = lax.axis_index("s")
    wid = sid * NC + cid
    zero16 = jnp.zeros((16,), jnp.float32)

    def zfill(t, carry):
        zero_v[t, :] = zero16
        return carry

    lax.fori_loop(0, K, zfill, 0)
    for j in range(RT // K):
        pltpu.sync_copy(zero_v, hist_sh.at[pl.ds(sid * RT + j * K, K)])
    pltpu.sync_copy(ones_hbm, ones_v)
    pltpu.sync_copy(dst_hbm.at[wid], idx_v)
    plsc.subcore_barrier()

    W = 8
    for c in range(W):
        pltpu.async_copy(ones_v, hist_sh.at[idx_v.at[c]], sem, add=True)

    def step(c, carry):
        pltpu.make_async_copy(ones_v, hist_sh.at[idx_v.at[0]], sem).wait()
        pltpu.async_copy(ones_v, hist_sh.at[idx_v.at[c + W]], sem, add=True)
        return carry

    lax.fori_loop(0, C - W, step, 0)
    for c in range(W):
        pltpu.make_async_copy(ones_v, hist_sh.at[idx_v.at[0]], sem).wait()
    plsc.subcore_barrier()
    pltpu.sync_copy(hist_sh.at[pl.ds(sid * RT, RT)],
                    out_hbm.at[cid, pl.ds(sid * RT, RT), :])


@functools.partial(
    pl.kernel,
    out_type=jax.ShapeDtypeStruct((NC, NP, D), jnp.float32),
    mesh=_mesh,
    compiler_params=pltpu.CompilerParams(use_tc_tiling_on_sc=False),
    scratch_types=[
        pltpu.VMEM((C2, K2), jnp.int32),
        pltpu.VMEM((C2, K2), jnp.int32),
        pltpu.VMEM((K2, D), jnp.float32),
        pltpu.VMEM((K2, D), jnp.float32),
        pltpu.VMEM((K2, D), jnp.float32),
        pltpu.VMEM((K2, D), jnp.float32),
        pltpu.VMEM_SHARED((NP, D), jnp.float32),
        pltpu.SemaphoreType.DMA,
        pltpu.SemaphoreType.DMA,
        pltpu.SemaphoreType.DMA,
        pltpu.SemaphoreType.DMA,
    ],
)
def _aggregate(src_hbm, dst_hbm, hs_hbm, out_hbm, src_v, dst_v,
               rows0, rows1, rows2, rows3, acc_sh, sem0, sem1, sem2, sem3):
    cid = lax.axis_index("c")
    sid = lax.axis_index("s")
    wid = sid * NC + cid
    @pl.when(cid == 0)
    def _():
        pltpu.sync_copy(hs_hbm.at[pl.ds(sid * RT, RT)],
                        acc_sh.at[pl.ds(sid * RT, RT)])

    @pl.when(cid == 1)
    def _():
        zero16 = jnp.zeros((16,), jnp.float32)

        def zfill(t, carry):
            rows0[t // 8, pl.ds((t % 8) * 16, 16)] = zero16
            return carry

        lax.fori_loop(0, K2 * 8, zfill, 0)
        for j in range(RT // K2):
            pltpu.sync_copy(rows0, acc_sh.at[pl.ds(sid * RT + j * K2, K2)])
        rem = RT - (RT // K2) * K2
        if rem:
            pltpu.sync_copy(rows0.at[pl.ds(0, rem)],
                            acc_sh.at[pl.ds(sid * RT + RT - rem, rem)])

    pltpu.sync_copy(src_hbm.at[wid], src_v)
    pltpu.sync_copy(dst_hbm.at[wid], dst_v)
    plsc.subcore_barrier()

    def gstart(c, rows, sem):
        pltpu.async_copy(hs_hbm.at[src_v.at[c]], rows, sem)

    def gwait(c, rows, sem):
        pltpu.make_async_copy(hs_hbm.at[src_v.at[c]], rows, sem).wait()

    def scat(c, rows):
        pltpu.sync_copy(rows, acc_sh.at[dst_v.at[c]], add=True)

    bufs = ((rows0, sem0), (rows1, sem1), (rows2, sem2), (rows3, sem3))
    RD = len(bufs)
    for j, (rows, sem) in enumerate(bufs):
        gstart(j, rows, sem)

    def step(i, carry):
        c = RD * i
        for j, (rows, sem) in enumerate(bufs):
            gwait(c + j, rows, sem)

            @pl.when(c + j + RD < C2)
            def _():
                gstart(c + j + RD, rows, sem)

            scat(c + j, rows)
        return carry

    lax.fori_loop(0, C2 // RD, step, 0)
    plsc.subcore_barrier()
    pltpu.sync_copy(acc_sh.at[pl.ds(sid * RT, RT)],
                    out_hbm.at[cid, pl.ds(sid * RT, RT), :])


BN = 5000


def _scaled_mm_body(x_ref, w_ref, b_ref, h_ref, out_ref):
    h = h_ref[...]
    deg = h[0, :, 0] + h[1, :, 0] + 1.0
    dis = lax.rsqrt(deg)
    y = jnp.dot(x_ref[...], w_ref[...], preferred_element_type=jnp.float32)
    out_ref[...] = dis[:, None] * (y + b_ref[...])


def _epilogue_body(a0_ref, a1_ref, h_ref, out_ref):
    h = h_ref[...]
    deg = h[0, :, 0] + h[1, :, 0] + 1.0
    dis = lax.rsqrt(deg)
    s = a0_ref[...] + a1_ref[...]
    out_ref[...] = jnp.maximum(dis[:, None] * s, 0.0)


def kernel(node_feats, adj, is_training, w, b):
    del is_training
    padh = PAD_NODE + (jnp.arange(EPADH, dtype=jnp.int32) % (NP - N))
    dst3 = jnp.concatenate([adj[1], padh]).reshape(NW, C, K)
    pad = PAD_NODE + (jnp.arange(EPAD, dtype=jnp.int32) % (NP - N))
    srcp = jnp.concatenate([adj[0], pad]).reshape(NW, C2, K2)
    dstp = jnp.concatenate([adj[1], pad]).reshape(NW, C2, K2)
    ones = jnp.ones((K, HW), jnp.float32)

    hist = _degree_hist(dst3, ones)

    hs = pl.pallas_call(
        _scaled_mm_body,
        grid=(N // BN,),
        in_specs=[
            pl.BlockSpec((BN, D), lambda i: (i, 0)),
            pl.BlockSpec((D, D), lambda i: (0, 0)),
            pl.BlockSpec((1, D), lambda i: (0, 0)),
            pl.BlockSpec((NC, BN, HW), lambda i: (0, i, 0)),
        ],
        out_specs=pl.BlockSpec((BN, D), lambda i: (i, 0)),
        out_shape=jax.ShapeDtypeStruct((NP, D), jnp.float32),
    )(node_feats, w, b.reshape(1, D), hist)

    acc = _aggregate(srcp, dstp, hs)

    out = pl.pallas_call(
        _epilogue_body,
        grid=(N // BN,),
        in_specs=[
            pl.BlockSpec((BN, D), lambda i: (i, 0)),
            pl.BlockSpec((BN, D), lambda i: (i, 0)),
            pl.BlockSpec((NC, BN, HW), lambda i: (0, i, 0)),
        ],
        out_specs=pl.BlockSpec((BN, D), lambda i: (i, 0)),
        out_shape=jax.ShapeDtypeStruct((N, D), jnp.float32),
    )(acc[0], acc[1], hist)
    return out
